# baseline (speedup 1.0000x reference)
import functools

import jax
import jax.numpy as jnp
from jax import lax
from jax.experimental import pallas as pl
from jax.experimental.pallas import tpu as pltpu
from jax.experimental.pallas import tpu_sc as plsc

B = 16384
F = 26
D = 16
FIELD = 100000
TOTAL = F * FIELD

NW = 32
SPW = B // NW
FPAD = 32
SPC = 128 // FPAD
NCHUNK = SPW // SPC
LROWS_PER_W = NCHUNK
LIN_BLK = 64
LIN_ROWS_PER_W = SPW * F // 128


def _sc_body(line_hbm, sub_hbm, lval_hbm, emb_hbm, cross_hbm, lsum_hbm,
             idx_v, sub_v, linv_v, lines0, lines1, cross_v, lsum_v,
             sem0, sem1):
    c = lax.axis_index("c")
    s = lax.axis_index("s")
    wid = s * 2 + c

    pltpu.sync_copy(line_hbm.at[pl.ds(wid * LROWS_PER_W, LROWS_PER_W)], idx_v)
    pltpu.sync_copy(sub_hbm.at[pl.ds(wid * LROWS_PER_W, LROWS_PER_W)], sub_v)
    pltpu.sync_copy(lval_hbm.at[pl.ds(wid * LIN_ROWS_PER_W, LIN_ROWS_PER_W)],
                    linv_v)

    bufs = (lines0, lines1)
    sems = (sem0, sem1)

    def emb_cp(b, buf):
        return pltpu.make_async_copy(
            emb_hbm.at[idx_v.at[b]], bufs[buf], sems[buf])

    emb_cp(0, 0).start()
    emb_cp(1, 1).start()

    def process(k, b, buf, odd):
        emb_cp(b, buf).wait()
        lines = bufs[buf]
        for i in range(SPC):
            base = i * FPAD
            sub_lo = sub_v[b, pl.ds(base, 16)]
            sub_hi = sub_v[b, pl.ds(base + 16, 16)]
            r = lines[base, pl.ds(sub_lo[0] * D, D)]
            s_acc = r
            q_acc = r * r
            for f in range(1, F):
                sub = sub_lo[f] if f < 16 else sub_hi[f - 16]
                r = lines[base + f, pl.ds(sub * D, D)]
                s_acc = s_acc + r
                q_acc = q_acc + r * r
            cross_v[k, pl.ds((4 * odd + i) * D, D)] = (
                0.5 * (s_acc * s_acc - q_acc))

        @pl.when(b + 2 < NCHUNK)
        def _():
            emb_cp(b + 2, buf).start()

    def body(k, carry):
        process(k, 2 * k, 0, 0)
        process(k, 2 * k + 1, 1, 1)
        return carry

    lax.fori_loop(0, NCHUNK // 2, body, 0, unroll=False)

    def lin_body(blk2, carry):
        for par in range(2):
            base_row = (blk2 * 2 + par) * (LIN_BLK * F // 128)
            for g in range(LIN_BLK // 16):
                o = g * 16
                acc = linv_v[base_row + o // 128, pl.ds(o % 128, 16)]
                for f in range(1, F):
                    o = f * LIN_BLK + g * 16
                    acc = acc + linv_v[base_row + o // 128, pl.ds(o % 128, 16)]
                lsum_v[blk2, pl.ds(par * LIN_BLK + g * 16, 16)] = acc
        return carry

    lax.fori_loop(0, SPW // (2 * LIN_BLK), lin_body, 0, unroll=False)

    pltpu.sync_copy(cross_v,
                    cross_hbm.at[pl.ds(wid * (SPW * D // 128), SPW * D // 128)])
    pltpu.sync_copy(lsum_v, lsum_hbm.at[pl.ds(wid * (SPW // 128), SPW // 128)])


_sc_gather = functools.partial(
    pl.kernel,
    mesh=plsc.VectorSubcoreMesh(core_axis_name="c", subcore_axis_name="s"),
    out_type=[
        jax.ShapeDtypeStruct((B * D // 128, 128), jnp.float32),
        jax.ShapeDtypeStruct((B // 128, 128), jnp.float32),
    ],
    scratch_types=[
        pltpu.VMEM((LROWS_PER_W, 128), jnp.int32),
        pltpu.VMEM((LROWS_PER_W, 128), jnp.int32),
        pltpu.VMEM((LIN_ROWS_PER_W, 128), jnp.float32),
        pltpu.VMEM((128, 128), jnp.float32),
        pltpu.VMEM((128, 128), jnp.float32),
        pltpu.VMEM((SPW * D // 128, 128), jnp.float32),
        pltpu.VMEM((SPW // 128, 128), jnp.float32),
        pltpu.SemaphoreType.DMA,
        pltpu.SemaphoreType.DMA,
    ],
    compiler_params=pltpu.CompilerParams(use_tc_tiling_on_sc=True),
)(_sc_body)


def _bn(h, ones_row, g, b, eps=1e-5):
    m = jnp.dot(ones_row, h, preferred_element_type=jnp.float32)
    ms = jnp.dot(ones_row, h * h, preferred_element_type=jnp.float32)
    scale = g * lax.rsqrt(ms - m * m + eps)
    shift = b - m * scale
    return h * scale + shift


def _mlp_body(cross_ref, lsum_ref, g0_ref, b0_ref, w1_ref, b1_ref, g1_ref,
              be1_ref, w2_ref, b2_ref, g2_ref, be2_ref, w3_ref, b3_ref,
              bias_ref, out_ref):
    ones_row = jnp.full((1, B), 1.0 / B, dtype=jnp.float32)
    cross = _bn(cross_ref[...], ones_row, g0_ref[...], b0_ref[...])
    h = jnp.dot(cross, w1_ref[...], preferred_element_type=jnp.float32)
    h = jnp.maximum(_bn(h + b1_ref[...], ones_row, g1_ref[...], be1_ref[...]), 0.0)
    h = jnp.dot(h, w2_ref[...], preferred_element_type=jnp.float32)
    h = jnp.maximum(_bn(h + b2_ref[...], ones_row, g2_ref[...], be2_ref[...]), 0.0)
    mlp = jnp.dot(h, w3_ref[...], preferred_element_type=jnp.float32)
    out_ref[...] = mlp + b3_ref[...] + lsum_ref[...] + bias_ref[...]


def kernel(x, emb_table, lin_table, lin_bias, bn0_gamma, bn0_beta,
           W1, b1, g1, be1, W2, b2, g2, be2, W3, b3):
    offsets = (jnp.arange(F, dtype=x.dtype) * FIELD)[None, :]
    xi = (x + offsets).astype(jnp.int32)
    xi_pad = jnp.concatenate([xi, xi[:, :FPAD - F]], axis=1)
    line_rows = (xi_pad >> 3).reshape(B * FPAD // 128, 128)
    sub_rows = (xi_pad & 7).reshape(B * FPAD // 128, 128)
    one = lax.optimization_barrier(jnp.float32(1.0))
    emb2 = (emb_table * one).reshape(TOTAL // 8, 128)

    xi_t = (xi.reshape(NW, SPW // LIN_BLK, LIN_BLK, F)
            .transpose(0, 1, 3, 2)
            .reshape(-1))
    lvals = jnp.take(lin_table, xi_t, axis=0, mode="clip")
    lvals = lvals.reshape(B * F // 128, 128)

    cross2, lsum2 = _sc_gather(line_rows, sub_rows, lvals, emb2)
    cross = cross2.reshape(B, D)
    lsum = lsum2.reshape(B)

    out = pl.pallas_call(
        _mlp_body,
        out_shape=jax.ShapeDtypeStruct((B, 1), jnp.float32),
    )(
        cross, lsum.reshape(B, 1),
        bn0_gamma.reshape(1, D), bn0_beta.reshape(1, D),
        W1, b1.reshape(1, -1), g1.reshape(1, -1), be1.reshape(1, -1),
        W2, b2.reshape(1, -1), g2.reshape(1, -1), be2.reshape(1, -1),
        W3, b3.reshape(1, 1), lin_bias.reshape(1, 1),
    )
    return out


# speedup vs baseline: 1.1600x; 1.1600x over previous
import functools

import jax
import jax.numpy as jnp
from jax import lax
from jax.experimental import pallas as pl
from jax.experimental.pallas import tpu as pltpu
from jax.experimental.pallas import tpu_sc as plsc

B = 16384
F = 26
D = 16
FIELD = 100000

NW = 32
SPW = B // NW
BLK = 64
NBLK = SPW // BLK
IDX_PER_BLK = BLK * F
ROWS_PER_BLK = IDX_PER_BLK // 128
IDX_ROWS_PER_W = SPW * F // 128


def _sc_body(xi_hbm, lval_hbm, emb_hbm, cross_hbm, lsum_hbm,
             idx_v, rows_v, linv_v, cross_v, lsum_v, sem):
    c = lax.axis_index("c")
    s = lax.axis_index("s")
    wid = s * 2 + c

    pltpu.sync_copy(xi_hbm.at[pl.ds(wid * IDX_ROWS_PER_W, IDX_ROWS_PER_W)], idx_v)
    pltpu.sync_copy(lval_hbm.at[pl.ds(wid * IDX_ROWS_PER_W, IDX_ROWS_PER_W)], linv_v)

    def blk_body(blk, carry):
        base_row = blk * ROWS_PER_BLK
        copies = []
        for j in range(ROWS_PER_BLK):
            cp = pltpu.make_async_copy(
                emb_hbm.at[idx_v.at[base_row + j]],
                rows_v.at[pl.ds(j * 128, 128)], sem)
            cp.start()
            copies.append(cp)
        for cp in copies:
            cp.wait()

        def samp_body(i, carry2):
            r = rows_v[i * F, :]
            s_acc = r
            q_acc = r * r
            for f in range(1, F):
                r = rows_v[i * F + f, :]
                s_acc = s_acc + r
                q_acc = q_acc + r * r
            cross_v[i, :] = 0.5 * (s_acc * s_acc - q_acc)
            return carry2

        lax.fori_loop(0, BLK, samp_body, 0, unroll=False)

        for g in range(BLK // 16):
            o = g * 16
            acc = linv_v[base_row + o // 128, pl.ds(o % 128, 16)]
            for f in range(1, F):
                o = f * BLK + g * 16
                acc = acc + linv_v[base_row + o // 128, pl.ds(o % 128, 16)]
            lsum_v[pl.ds(g * 16, 16)] = acc

        out_base = wid * SPW + blk * BLK
        pltpu.sync_copy(cross_v, cross_hbm.at[pl.ds(out_base, BLK)])
        pltpu.sync_copy(lsum_v, lsum_hbm.at[pl.ds(out_base, BLK)])
        return carry

    lax.fori_loop(0, NBLK, blk_body, 0, unroll=False)


_sc_gather = functools.partial(
    pl.kernel,
    mesh=plsc.VectorSubcoreMesh(core_axis_name="c", subcore_axis_name="s"),
    out_type=[
        jax.ShapeDtypeStruct((B, D), jnp.float32),
        jax.ShapeDtypeStruct((B,), jnp.float32),
    ],
    scratch_types=[
        pltpu.VMEM((IDX_ROWS_PER_W, 128), jnp.int32),
        pltpu.VMEM((IDX_PER_BLK, D), jnp.float32),
        pltpu.VMEM((IDX_ROWS_PER_W, 128), jnp.float32),
        pltpu.VMEM((BLK, D), jnp.float32),
        pltpu.VMEM((BLK,), jnp.float32),
        pltpu.SemaphoreType.DMA,
    ],
    compiler_params=pltpu.CompilerParams(use_tc_tiling_on_sc=False),
)(_sc_body)


def _bn(h, ones_row, g, b, eps=1e-5):
    m = jnp.dot(ones_row, h, preferred_element_type=jnp.float32)
    ms = jnp.dot(ones_row, h * h, preferred_element_type=jnp.float32)
    scale = g * lax.rsqrt(ms - m * m + eps)
    shift = b - m * scale
    return h * scale + shift


def _mlp_body(cross_ref, lsum_ref, g0_ref, b0_ref, w1_ref, b1_ref, g1_ref,
              be1_ref, w2_ref, b2_ref, g2_ref, be2_ref, w3_ref, b3_ref,
              bias_ref, out_ref):
    ones_row = jnp.full((1, B), 1.0 / B, dtype=jnp.float32)
    cross = _bn(cross_ref[...], ones_row, g0_ref[...], b0_ref[...])
    h = jnp.dot(cross, w1_ref[...], preferred_element_type=jnp.float32)
    h = jnp.maximum(_bn(h + b1_ref[...], ones_row, g1_ref[...], be1_ref[...]), 0.0)
    h = jnp.dot(h, w2_ref[...], preferred_element_type=jnp.float32)
    h = jnp.maximum(_bn(h + b2_ref[...], ones_row, g2_ref[...], be2_ref[...]), 0.0)
    mlp = jnp.dot(h, w3_ref[...], preferred_element_type=jnp.float32)
    out_ref[...] = mlp + b3_ref[...] + lsum_ref[...] + bias_ref[...]


def kernel(x, emb_table, lin_table, lin_bias, bn0_gamma, bn0_beta,
           W1, b1, g1, be1, W2, b2, g2, be2, W3, b3):
    offsets = (jnp.arange(F, dtype=x.dtype) * FIELD)[None, :]
    xi = (x + offsets).astype(jnp.int32)
    xi_rows = xi.reshape(B * F // 128, 128)
    xi_t = (xi.reshape(NW, NBLK, BLK, F)
            .transpose(0, 1, 3, 2)
            .reshape(-1))
    lvals = jnp.take(lin_table, xi_t, axis=0, mode="clip")
    lvals = lvals.reshape(B * F // 128, 128)

    cross, lsum = _sc_gather(xi_rows, lvals, emb_table)

    out = pl.pallas_call(
        _mlp_body,
        out_shape=jax.ShapeDtypeStruct((B, 1), jnp.float32),
    )(
        cross, lsum.reshape(B, 1),
        bn0_gamma.reshape(1, D), bn0_beta.reshape(1, D),
        W1, b1.reshape(1, -1), g1.reshape(1, -1), be1.reshape(1, -1),
        W2, b2.reshape(1, -1), g2.reshape(1, -1), be2.reshape(1, -1),
        W3, b3.reshape(1, 1), lin_bias.reshape(1, 1),
    )
    return out


# speedup vs baseline: 1.1749x; 1.0129x over previous
import functools

import jax
import jax.numpy as jnp
from jax import lax
from jax.experimental import pallas as pl
from jax.experimental.pallas import tpu as pltpu
from jax.experimental.pallas import tpu_sc as plsc

B = 16384
F = 26
D = 16
FIELD = 100000

NW = 32
SPW = B // NW
BLK = 64
NBLK = SPW // BLK
IDX_PER_BLK = BLK * F
ROWS_PER_BLK = IDX_PER_BLK // 128
IDX_ROWS_PER_W = SPW * F // 128


def _sc_body(xi_hbm, lval_hbm, emb_hbm, cross_hbm, lsum_hbm,
             idx_v, rows0, rows1, linv_v, cross_v, lsum_v, sem0, sem1):
    c = lax.axis_index("c")
    s = lax.axis_index("s")
    wid = s * 2 + c

    pltpu.sync_copy(xi_hbm.at[pl.ds(wid * IDX_ROWS_PER_W, IDX_ROWS_PER_W)], idx_v)
    pltpu.sync_copy(lval_hbm.at[pl.ds(wid * IDX_ROWS_PER_W, IDX_ROWS_PER_W)], linv_v)

    bufs = (rows0, rows1)
    sems = (sem0, sem1)

    def fire(blk, buf):
        copies = []
        for j in range(ROWS_PER_BLK):
            cp = pltpu.make_async_copy(
                emb_hbm.at[idx_v.at[blk * ROWS_PER_BLK + j]],
                bufs[buf].at[pl.ds(j * 128, 128)], sems[buf])
            cp.start()
            copies.append(cp)
        return copies

    def drain(blk, buf):
        for j in range(ROWS_PER_BLK):
            pltpu.make_async_copy(
                emb_hbm.at[idx_v.at[blk * ROWS_PER_BLK + j]],
                bufs[buf].at[pl.ds(j * 128, 128)], sems[buf]).wait()

    fire(0, 0)
    fire(1, 1)

    def process(blk, buf):
        drain(blk, buf)
        rows_v = bufs[buf]

        def samp_body(i, carry2):
            r = rows_v[i * F, :]
            s_acc = r
            q_acc = r * r
            for f in range(1, F):
                r = rows_v[i * F + f, :]
                s_acc = s_acc + r
                q_acc = q_acc + r * r
            cross_v[i, :] = 0.5 * (s_acc * s_acc - q_acc)
            return carry2

        lax.fori_loop(0, BLK, samp_body, 0, unroll=False)

        @pl.when(blk + 2 < NBLK)
        def _():
            fire(blk + 2, buf)

        base_row = blk * ROWS_PER_BLK
        for g in range(BLK // 16):
            o = g * 16
            acc = linv_v[base_row + o // 128, pl.ds(o % 128, 16)]
            for f in range(1, F):
                o = f * BLK + g * 16
                acc = acc + linv_v[base_row + o // 128, pl.ds(o % 128, 16)]
            lsum_v[pl.ds(g * 16, 16)] = acc

        out_base = wid * SPW + blk * BLK
        pltpu.sync_copy(cross_v, cross_hbm.at[pl.ds(out_base, BLK)])
        pltpu.sync_copy(lsum_v, lsum_hbm.at[pl.ds(out_base, BLK)])

    def blk_body(k, carry):
        process(2 * k, 0)
        process(2 * k + 1, 1)
        return carry

    lax.fori_loop(0, NBLK // 2, blk_body, 0, unroll=False)


_sc_gather = functools.partial(
    pl.kernel,
    mesh=plsc.VectorSubcoreMesh(core_axis_name="c", subcore_axis_name="s"),
    out_type=[
        jax.ShapeDtypeStruct((B, D), jnp.float32),
        jax.ShapeDtypeStruct((B,), jnp.float32),
    ],
    scratch_types=[
        pltpu.VMEM((IDX_ROWS_PER_W, 128), jnp.int32),
        pltpu.VMEM((IDX_PER_BLK, D), jnp.float32),
        pltpu.VMEM((IDX_PER_BLK, D), jnp.float32),
        pltpu.VMEM((IDX_ROWS_PER_W, 128), jnp.float32),
        pltpu.VMEM((BLK, D), jnp.float32),
        pltpu.VMEM((BLK,), jnp.float32),
        pltpu.SemaphoreType.DMA,
        pltpu.SemaphoreType.DMA,
    ],
    compiler_params=pltpu.CompilerParams(use_tc_tiling_on_sc=False),
)(_sc_body)


def _bn(h, ones_row, g, b, eps=1e-5):
    m = jnp.dot(ones_row, h, preferred_element_type=jnp.float32)
    ms = jnp.dot(ones_row, h * h, preferred_element_type=jnp.float32)
    scale = g * lax.rsqrt(ms - m * m + eps)
    shift = b - m * scale
    return h * scale + shift


def _mlp_body(cross_ref, lsum_ref, g0_ref, b0_ref, w1_ref, b1_ref, g1_ref,
              be1_ref, w2_ref, b2_ref, g2_ref, be2_ref, w3_ref, b3_ref,
              bias_ref, out_ref):
    ones_row = jnp.full((1, B), 1.0 / B, dtype=jnp.float32)
    cross = _bn(cross_ref[...], ones_row, g0_ref[...], b0_ref[...])
    h = jnp.dot(cross, w1_ref[...], preferred_element_type=jnp.float32)
    h = jnp.maximum(_bn(h + b1_ref[...], ones_row, g1_ref[...], be1_ref[...]), 0.0)
    h = jnp.dot(h, w2_ref[...], preferred_element_type=jnp.float32)
    h = jnp.maximum(_bn(h + b2_ref[...], ones_row, g2_ref[...], be2_ref[...]), 0.0)
    mlp = jnp.dot(h, w3_ref[...], preferred_element_type=jnp.float32)
    out_ref[...] = mlp + b3_ref[...] + lsum_ref[...] + bias_ref[...]


def kernel(x, emb_table, lin_table, lin_bias, bn0_gamma, bn0_beta,
           W1, b1, g1, be1, W2, b2, g2, be2, W3, b3):
    offsets = (jnp.arange(F, dtype=x.dtype) * FIELD)[None, :]
    xi = (x + offsets).astype(jnp.int32)
    xi_rows = xi.reshape(B * F // 128, 128)
    xi_t = (xi.reshape(NW, NBLK, BLK, F)
            .transpose(0, 1, 3, 2)
            .reshape(-1))
    lvals = jnp.take(lin_table, xi_t, axis=0, mode="clip")
    lvals = lvals.reshape(B * F // 128, 128)

    cross, lsum = _sc_gather(xi_rows, lvals, emb_table)

    out = pl.pallas_call(
        _mlp_body,
        out_shape=jax.ShapeDtypeStruct((B, 1), jnp.float32),
    )(
        cross, lsum.reshape(B, 1),
        bn0_gamma.reshape(1, D), bn0_beta.reshape(1, D),
        W1, b1.reshape(1, -1), g1.reshape(1, -1), be1.reshape(1, -1),
        W2, b2.reshape(1, -1), g2.reshape(1, -1), be2.reshape(1, -1),
        W3, b3.reshape(1, 1), lin_bias.reshape(1, 1),
    )
    return out
